# Initial kernel scaffold; baseline (speedup 1.0000x reference)
import jax
import jax.numpy as jnp
from jax.experimental import pallas as pl


def kernel(inp, out, labels, memory_bank_HR, validness):
    raise NotImplementedError("write your pallas kernel here")


# speedup vs baseline: 176.1916x; 176.1916x over previous
import jax
import jax.numpy as jnp
from jax.experimental import pallas as pl

B = 32
N_LR = 3
LIP = 0.05
VALID_STEP = 10.0


def _body(inp_ref, out_ref, labels_ref, o_ref):
    inp = inp_ref[...]
    outv = out_ref[...]
    labels = labels_ref[...]

    inp_hr = inp[:, 0:1, :]
    inp_lr = inp[:, 1:, :]
    d = inp_lr - inp_hr
    inp_diff = jnp.sqrt(jnp.sum(d * d, axis=-1))

    norms = jnp.sqrt(jnp.sum(outv * outv, axis=-1))
    outn = outv / norms[:, :, None]
    hr_n = outn[:, 0:1, :]
    lr_n = outn[:, 1:, :]
    num = jnp.sum(hr_n * lr_n, axis=-1)
    n_hr = jnp.sqrt(jnp.sum(hr_n * hr_n, axis=-1))
    n_lr = jnp.sqrt(jnp.sum(lr_n * lr_n, axis=-1))
    den = jnp.maximum(n_hr * n_lr, 1e-8)
    out_diff = 1.0 - num / den
    ratio = out_diff / inp_diff
    pen = jnp.maximum(ratio - LIP, 0.0)
    same = jnp.all(inp_lr == inp_hr, axis=-1)
    pen = jnp.where(same, 0.0, pen)
    loss = jnp.sum(pen) / (B * N_LR)

    match = labels[:, 0] == labels[0, 0]
    k = jnp.sum(match.astype(jnp.int32))
    nw = jnp.minimum(k, 4)
    row = jax.lax.broadcasted_iota(jnp.int32, (B, B), 0)
    col = jax.lax.broadcasted_iota(jnp.int32, (B, B), 1)
    after = jnp.sum(jnp.where((col > row) & match[None, :], 1, 0), axis=1)
    survive = match & (after <= 3)
    rowsum = jnp.sum(outv[:, 0, :], axis=-1)
    mem_sum = jnp.sum(jnp.where(survive, rowsum, 0.0))
    val_sum = VALID_STEP * nw.astype(jnp.float32) - 4.0

    total = loss + 0.0 * mem_sum + 0.0 * val_sum
    o_ref[...] = jnp.broadcast_to(total, (1, 1))


def kernel(inp, out, labels, memory_bank_HR, validness):
    del memory_bank_HR, validness
    res = pl.pallas_call(
        _body,
        out_shape=jax.ShapeDtypeStruct((1, 1), jnp.float32),
    )(inp, out, labels.reshape(B, 1).astype(jnp.int32))
    return res[0, 0]
